# baseline (speedup 1.0000x reference)
import jax
import jax.numpy as jnp
from jax.experimental import pallas as pl
from jax.experimental.pallas import tpu as pltpu

_C = 80


def _ohem_kernel(x_ref, t_ref, out_ref, rs_vm, tot_sum, nr_fg):
    i = pl.program_id(0)
    nblk = pl.num_programs(0)
    B = t_ref.shape[0]

    @pl.when(i == 0)
    def _init():
        tot_sum[0] = jnp.float32(0.0)
        nr_fg[0] = jnp.int32(0)

    x = x_ref[...]
    t = t_ref[...]
    cls = jax.lax.broadcasted_iota(jnp.int32, (_C, B), 0) + 1
    z = t[None, :] == cls
    _LOG2E = 1.4426950408889634
    _LN2 = 0.6931471805599453
    w = jnp.where(z, jnp.float32(-_LOG2E), jnp.float32(_LOG2E))
    le = jnp.log2(1.0 + jnp.exp2(x * w))
    rs = jnp.sum(le, axis=0) * jnp.float32(_LN2)
    is_bg = t == 0
    rs_vm[i, :] = jnp.where(is_bg, rs, 0.0)
    tot_sum[0] += jnp.sum(rs)
    nr_fg[0] += jnp.sum((t > 0).astype(jnp.int32))

    @pl.when(i == nblk - 1)
    def _finish():
        n_total = nblk * B
        k = jnp.maximum(jnp.int32(128),
                        jnp.minimum(n_total - nr_fg[0], nr_fg[0] * 3))

        def body(j, prefix):
            cand = prefix | jax.lax.shift_left(jnp.int32(1), 30 - j)
            candf = jax.lax.bitcast_convert_type(cand, jnp.float32)
            cnt = jnp.sum((rs_vm[...] >= candf).astype(jnp.int32))
            return jnp.where(cnt >= k, cand, prefix)

        prefix = jax.lax.fori_loop(0, 31, body, jnp.int32(0))
        thr = jax.lax.bitcast_convert_type(prefix, jnp.float32)
        v = rs_vm[...]
        gt = v > thr
        cnt_gt = jnp.sum(gt.astype(jnp.int32))
        sum_gt = jnp.sum(jnp.where(gt, v, 0.0))
        train_bg = sum_gt + (k - cnt_gt).astype(jnp.float32) * thr
        bg_total = jnp.sum(v)
        fg_sum = tot_sum[0] - bg_total
        out_ref[0] = (fg_sum + train_bg) * 0.25


def kernel(logits, targets):
    N, C = logits.shape
    xt = logits.T
    B = 32768
    nblk = N // B
    out = pl.pallas_call(
        _ohem_kernel,
        grid=(nblk,),
        in_specs=[
            pl.BlockSpec((C, B), lambda i: (0, i)),
            pl.BlockSpec((B,), lambda i: (i,)),
        ],
        out_specs=pl.BlockSpec(memory_space=pltpu.SMEM),
        out_shape=jax.ShapeDtypeStruct((1,), jnp.float32),
        scratch_shapes=[
            pltpu.VMEM((N // B, B), jnp.float32),
            pltpu.SMEM((1,), jnp.float32),
            pltpu.SMEM((1,), jnp.int32),
        ],
        compiler_params=pltpu.CompilerParams(
            dimension_semantics=("arbitrary",),
        ),
    )(xt, targets)
    return out[0]


# speedup vs baseline: 1.0417x; 1.0417x over previous
import jax
import jax.numpy as jnp
from jax.experimental import pallas as pl
from jax.experimental.pallas import tpu as pltpu

_C = 80


def _ohem_kernel(x_ref, t_ref, out_ref, rs_vm, tot_sum, nr_fg):
    i = pl.program_id(0)
    nblk = pl.num_programs(0)
    B = t_ref.shape[0]

    @pl.when(i == 0)
    def _init():
        tot_sum[0] = jnp.float32(0.0)
        nr_fg[0] = jnp.int32(0)

    x = x_ref[...]
    t = t_ref[...]
    cls = jax.lax.broadcasted_iota(jnp.int32, (_C, B), 0) + 1
    z = t[None, :] == cls
    _LOG2E = 1.4426950408889634
    _LN2 = 0.6931471805599453
    w = jnp.where(z, jnp.float32(-_LOG2E), jnp.float32(_LOG2E))
    le = jnp.log2(1.0 + jnp.exp2(x * w))
    rs = jnp.sum(le, axis=0) * jnp.float32(_LN2)
    is_bg = t == 0
    rs_vm[i, :] = jnp.where(is_bg, rs, 0.0)
    tot_sum[0] += jnp.sum(rs)
    nr_fg[0] += jnp.sum((t > 0).astype(jnp.int32))

    @pl.when(i == nblk - 1)
    def _finish():
        n_total = nblk * B
        k = jnp.maximum(jnp.int32(128),
                        jnp.minimum(n_total - nr_fg[0], nr_fg[0] * 3))

        def body(j, prefix):
            cand = prefix | jax.lax.shift_left(jnp.int32(1), 30 - j)
            candf = jax.lax.bitcast_convert_type(cand, jnp.float32)
            cnt = jnp.sum((rs_vm[...] >= candf).astype(jnp.int32))
            return jnp.where(cnt >= k, cand, prefix)

        prefix = jax.lax.fori_loop(0, 31, body, jnp.int32(0))
        thr = jax.lax.bitcast_convert_type(prefix, jnp.float32)
        v = rs_vm[...]
        gt = v > thr
        cnt_gt = jnp.sum(gt.astype(jnp.int32))
        sum_gt = jnp.sum(jnp.where(gt, v, 0.0))
        train_bg = sum_gt + (k - cnt_gt).astype(jnp.float32) * thr
        bg_total = jnp.sum(v)
        fg_sum = tot_sum[0] - bg_total
        out_ref[0] = (fg_sum + train_bg) * 0.25


def kernel(logits, targets):
    N, C = logits.shape
    xt = logits.T
    B = 16384
    nblk = N // B
    out = pl.pallas_call(
        _ohem_kernel,
        grid=(nblk,),
        in_specs=[
            pl.BlockSpec((C, B), lambda i: (0, i)),
            pl.BlockSpec((B,), lambda i: (i,)),
        ],
        out_specs=pl.BlockSpec(memory_space=pltpu.SMEM),
        out_shape=jax.ShapeDtypeStruct((1,), jnp.float32),
        scratch_shapes=[
            pltpu.VMEM((N // B, B), jnp.float32),
            pltpu.SMEM((1,), jnp.float32),
            pltpu.SMEM((1,), jnp.int32),
        ],
        compiler_params=pltpu.CompilerParams(
            dimension_semantics=("arbitrary",),
        ),
    )(xt, targets)
    return out[0]


# speedup vs baseline: 1.1288x; 1.0836x over previous
import jax
import jax.numpy as jnp
from jax.experimental import pallas as pl
from jax.experimental.pallas import tpu as pltpu

_C = 80


def _ohem_kernel(x_ref, t_ref, out_ref, rs_vm, tot_sum, nr_fg):
    i = pl.program_id(0)
    nblk = pl.num_programs(0)
    B = t_ref.shape[0]

    @pl.when(i == 0)
    def _init():
        tot_sum[0] = jnp.float32(0.0)
        nr_fg[0] = jnp.int32(0)

    x = x_ref[...]
    t = t_ref[...]
    cls = jax.lax.broadcasted_iota(jnp.int32, (_C, B), 0) + 1
    z = t[None, :] == cls
    _LOG2E = 1.4426950408889634
    _LN2 = 0.6931471805599453
    w = jnp.where(z, jnp.float32(-_LOG2E), jnp.float32(_LOG2E))
    le = jnp.log2(1.0 + jnp.exp2(x * w))
    rs = jnp.sum(le, axis=0) * jnp.float32(_LN2)
    is_bg = t == 0
    rs_vm[i, :] = jnp.where(is_bg, rs, 0.0)
    tot_sum[0] += jnp.sum(rs)
    nr_fg[0] += jnp.sum((t > 0).astype(jnp.int32))

    @pl.when(i == nblk - 1)
    def _finish():
        n_total = nblk * B
        k = jnp.maximum(jnp.int32(128),
                        jnp.minimum(n_total - nr_fg[0], nr_fg[0] * 3))

        c30 = jax.lax.bitcast_convert_type(jnp.int32(1 << 30), jnp.float32)
        cnt30 = jnp.sum((rs_vm[...] >= c30).astype(jnp.int32))
        prefix0 = jnp.where(cnt30 >= k, jnp.int32(1 << 30), jnp.int32(0))

        def body(j, prefix):
            step = jax.lax.shift_left(jnp.int32(1), 28 - 2 * j)
            c1 = jax.lax.bitcast_convert_type(prefix + step, jnp.float32)
            c2 = jax.lax.bitcast_convert_type(prefix + 2 * step, jnp.float32)
            c3 = jax.lax.bitcast_convert_type(prefix + 3 * step, jnp.float32)
            v = rs_vm[...]
            n1 = jnp.sum((v >= c1).astype(jnp.int32))
            n2 = jnp.sum((v >= c2).astype(jnp.int32))
            n3 = jnp.sum((v >= c3).astype(jnp.int32))
            sel = ((n1 >= k).astype(jnp.int32) + (n2 >= k).astype(jnp.int32)
                   + (n3 >= k).astype(jnp.int32))
            return prefix + step * sel

        prefix = jax.lax.fori_loop(0, 15, body, prefix0)
        thr = jax.lax.bitcast_convert_type(prefix, jnp.float32)
        v = rs_vm[...]
        gt = v > thr
        cnt_gt = jnp.sum(gt.astype(jnp.int32))
        sum_gt = jnp.sum(jnp.where(gt, v, 0.0))
        train_bg = sum_gt + (k - cnt_gt).astype(jnp.float32) * thr
        bg_total = jnp.sum(v)
        fg_sum = tot_sum[0] - bg_total
        out_ref[0] = (fg_sum + train_bg) * 0.25


def kernel(logits, targets):
    N, C = logits.shape
    xt = logits.T
    B = 16384
    nblk = N // B
    out = pl.pallas_call(
        _ohem_kernel,
        grid=(nblk,),
        in_specs=[
            pl.BlockSpec((C, B), lambda i: (0, i)),
            pl.BlockSpec((B,), lambda i: (i,)),
        ],
        out_specs=pl.BlockSpec(memory_space=pltpu.SMEM),
        out_shape=jax.ShapeDtypeStruct((1,), jnp.float32),
        scratch_shapes=[
            pltpu.VMEM((N // B, B), jnp.float32),
            pltpu.SMEM((1,), jnp.float32),
            pltpu.SMEM((1,), jnp.int32),
        ],
        compiler_params=pltpu.CompilerParams(
            dimension_semantics=("arbitrary",),
        ),
    )(xt, targets)
    return out[0]
